# baseline (speedup 1.0000x reference)
import functools

import jax
import jax.numpy as jnp
from jax import lax
from jax.experimental import pallas as pl
from jax.experimental.pallas import tpu as pltpu
from jax.experimental.pallas import tpu_sc as plsc

N = 50000
D = 128
E = 200000
NE = 3
L = 3

C = 4
CW = 32
N_PAD = 50048
ROWS_PER_TILE = N_PAD // 16

BS = 128
NBUF = 3
G = 34
NB = G * NBUF
E_PAD = 16 * NB * BS
PAD_IDX = N

_mesh = plsc.VectorSubcoreMesh(core_axis_name="c", subcore_axis_name="s")
_sc_params = pltpu.CompilerParams(use_tc_tiling_on_sc=False)


@functools.partial(
    pl.kernel,
    mesh=_mesh,
    compiler_params=_sc_params,
    out_type=[jax.ShapeDtypeStruct((2, N_PAD, CW), jnp.float32)
              for _ in range(NE)],
    scratch_types=[
        pltpu.VMEM((6, BS), jnp.int32),
        pltpu.VMEM((BS, CW), jnp.float32),
        pltpu.VMEM_SHARED((N_PAD, CW), jnp.float32),
    ],
)
def _sc_deg(i0, i1, i2, ones_h, zer_h, o0, o1, o2, idxv, onesv, acc):
    cid = lax.axis_index("c")
    sid = lax.axis_index("s")
    row_lo = sid * ROWS_PER_TILE
    pltpu.sync_copy(ones_h, onesv)
    for e, (iref, oref) in enumerate(((i0, o0), (i1, o1), (i2, o2))):
        pltpu.sync_copy(zer_h, acc.at[pl.ds(row_lo, ROWS_PER_TILE)])
        plsc.subcore_barrier()
        base_g = cid * (G // 2)

        def body(j, _, iref=iref):
            pltpu.sync_copy(iref.at[sid].at[base_g + j], idxv)
            for bb in range(NBUF):
                pltpu.sync_copy(onesv, acc.at[idxv.at[NBUF + bb]], add=True)
            return 0

        lax.fori_loop(0, G // 2, body, 0)
        plsc.subcore_barrier()
        for cval in range(2):
            @pl.when(cid == cval)
            def _(oref=oref, cval=cval):
                pltpu.sync_copy(acc.at[pl.ds(row_lo, ROWS_PER_TILE)],
                                oref.at[cval].at[pl.ds(row_lo, ROWS_PER_TILE)])
        plsc.subcore_barrier()


def _make_sc_agg(chunk_pair):
    @functools.partial(
        pl.kernel,
        mesh=plsc.VectorSubcoreMesh(core_axis_name="c", subcore_axis_name="s",
                                    num_cores=1),
        compiler_params=_sc_params,
        out_type=[jax.ShapeDtypeStruct((N_PAD, CW), jnp.float32)
                  for _ in range(NE * 2)],
        scratch_types=[
            pltpu.VMEM((12, BS), jnp.int32),
            pltpu.VMEM((2, NBUF, BS, CW), jnp.float32),
            pltpu.SemaphoreType.DMA,
            pltpu.SemaphoreType.DMA,
            pltpu.VMEM_SHARED((N_PAD, CW), jnp.float32),
        ],
    )
    def agg(ha, hb, i0, i1, i2, zer_h, *rest):
        _sc_agg_body(chunk_pair, (ha, hb), (i0, i1, i2), zer_h, rest)

    return agg


def _sc_agg_body(chunk_pair, hrefs, irefs, zer_h, rest):
    outs = rest[:NE * 2]
    idxv, rows, isem, gsem, acc = rest[NE * 2:]
    sid = lax.axis_index("s")
    row_lo = sid * ROWS_PER_TILE

    def one_pass(href, oref, iref):
        tidx = iref.at[sid]
        pltpu.sync_copy(zer_h, acc.at[pl.ds(row_lo, ROWS_PER_TILE)])
        plsc.subcore_barrier()

        def fetch_idx(g, p):
            pltpu.async_copy(tidx.at[g], idxv.at[pl.ds(6 * p, 6)], isem)

        def wait_idx(p):
            pltpu.make_async_copy(tidx.at[0], idxv.at[pl.ds(6 * p, 6)],
                                  isem).wait()

        def gathers(p):
            for bb in range(NBUF):
                pltpu.async_copy(href.at[idxv.at[6 * p + bb]],
                                 rows.at[p].at[bb], gsem)

        def drain_scatter(p):
            for bb in range(NBUF):
                pltpu.make_async_copy(href.at[idxv.at[0]], rows.at[p].at[bb],
                                      gsem).wait()
                pltpu.sync_copy(rows.at[p].at[bb],
                                acc.at[idxv.at[6 * p + NBUF + bb]], add=True)

        fetch_idx(0, 0)
        fetch_idx(1, 1)
        wait_idx(0)
        gathers(0)

        def outer(o, _):
            for p in range(2):
                g = 2 * o + p
                drain_scatter(p)
                fetch_idx(g + 2, p)
                wait_idx(1 - p)
                gathers(1 - p)
            return 0

        lax.fori_loop(0, G // 2 - 1, outer, 0)
        drain_scatter(0)
        wait_idx(1)
        gathers(1)
        drain_scatter(1)

        plsc.subcore_barrier()
        pltpu.sync_copy(acc.at[pl.ds(row_lo, ROWS_PER_TILE)],
                        oref.at[pl.ds(row_lo, ROWS_PER_TILE)])
        plsc.subcore_barrier()

    del chunk_pair
    for e, iref in enumerate(irefs):
        for k in range(2):
            one_pass(hrefs[k], outs[e * 2 + k], iref)


_SC_AGG_A = _make_sc_agg((0, 1))
_SC_AGG_B = _make_sc_agg((2, 3))


def _conv_body(h0, h1, h2, h3,
               a00, a01, a02, a03, a10, a11, a12, a13, a20, a21, a22, a23,
               g0, g1, g2, ws, wn, bs, out_ref, st_ref, sacc, *, act):
    i = pl.program_id(0)
    hb = (h0, h1, h2, h3)
    ab = ((a00, a01, a02, a03), (a10, a11, a12, a13), (a20, a21, a22, a23))
    db = (g0, g1, g2)
    acc = jnp.zeros((128, 128), jnp.float32)
    for e in range(NE):
        dg = db[e][...]
        deg = dg[0, :, 0:1] + dg[1, :, 0:1]
        inv = 1.0 / jnp.maximum(deg, 1.0)
        t = jnp.zeros((128, 128), jnp.float32)
        for c in range(C):
            t += jnp.dot(hb[c][...], ws[e, pl.ds(c * CW, CW), :],
                         preferred_element_type=jnp.float32)
            t += jnp.dot(ab[e][c][...] * inv, wn[e, pl.ds(c * CW, CW), :],
                         preferred_element_type=jnp.float32)
        t += bs[pl.ds(e, 1), :]
        if act:
            t = jnp.maximum(t, 0.0)
        acc += t
    rows = i * 128 + lax.broadcasted_iota(jnp.int32, (128, 1), 0)
    acc = jnp.where(rows < N, acc, 0.0)
    out_ref[...] = acc
    st = jnp.concatenate(
        [jnp.sum(acc, axis=0, keepdims=True),
         jnp.sum(acc * acc, axis=0, keepdims=True)], axis=0)

    @pl.when(i == 0)
    def _():
        sacc[...] = st

    @pl.when(i > 0)
    def _():
        sacc[...] += st

    @pl.when(i == N_PAD // 128 - 1)
    def _():
        st_ref[...] = sacc[...]


def _tc_conv(h_chunks, aggs, degs, ws, wn, bsum, act):
    nblk = N_PAD // 128
    cspec = pl.BlockSpec((128, CW), lambda i: (i, 0))
    dspec = pl.BlockSpec((2, 128, CW), lambda i: (0, i, 0))
    body = functools.partial(_conv_body, act=act)
    return pl.pallas_call(
        body,
        grid=(nblk,),
        in_specs=([cspec] * C + [cspec] * (NE * C) + [dspec] * NE
                  + [pl.BlockSpec((NE, 128, 128), lambda i: (0, 0, 0)),
                     pl.BlockSpec((NE, 128, 128), lambda i: (0, 0, 0)),
                     pl.BlockSpec((NE, 128), lambda i: (0, 0))]),
        out_specs=[pl.BlockSpec((128, 128), lambda i: (i, 0)),
                   pl.BlockSpec((2, 128), lambda i: (0, 0))],
        out_shape=[jax.ShapeDtypeStruct((N_PAD, 128), jnp.float32),
                   jax.ShapeDtypeStruct((2, 128), jnp.float32)],
        scratch_shapes=[pltpu.VMEM((2, 128), jnp.float32)],
    )(*h_chunks, *aggs, *degs, ws, wn, bsum)


def _bn_body(x_ref, st_ref, g_ref, b_ref, *out_refs, chunked):
    i = pl.program_id(0)
    st = st_ref[...]
    mean = st[0:1, :] / N
    var = st[1:2, :] / N - mean * mean
    scale = g_ref[...] / jnp.sqrt(var + 1e-5)
    shift = b_ref[...] - mean * scale
    y = x_ref[...] * scale + shift
    rows = i * 128 + lax.broadcasted_iota(jnp.int32, (128, 1), 0)
    y = jnp.where(rows < N, y, 0.0)
    if chunked:
        for c in range(C):
            out_refs[c][...] = y[:, c * CW:(c + 1) * CW]
    else:
        out_refs[0][...] = y


def _tc_bn(out, stats, g, b, chunked):
    nblk = N_PAD // 128
    if chunked:
        out_specs = [pl.BlockSpec((128, CW), lambda i: (i, 0))
                     for _ in range(C)]
        out_shape = [jax.ShapeDtypeStruct((N_PAD, CW), jnp.float32)
                     for _ in range(C)]
    else:
        out_specs = [pl.BlockSpec((128, 128), lambda i: (i, 0))]
        out_shape = [jax.ShapeDtypeStruct((N, 128), jnp.float32)]
    body = functools.partial(_bn_body, chunked=chunked)
    res = pl.pallas_call(
        body,
        grid=(nblk,),
        in_specs=[pl.BlockSpec((128, 128), lambda i: (i, 0)),
                  pl.BlockSpec((2, 128), lambda i: (0, 0)),
                  pl.BlockSpec((1, 128), lambda i: (0, 0)),
                  pl.BlockSpec((1, 128), lambda i: (0, 0))],
        out_specs=out_specs,
        out_shape=out_shape,
    )(out, stats, g, b)
    return res


def _prep_edge(ei):
    pad = E_PAD - E
    src = jnp.concatenate([ei[0], jnp.full((pad,), PAD_IDX, jnp.int32)])
    dst = jnp.concatenate([ei[1], jnp.full((pad,), PAD_IDX, jnp.int32)])
    src = src.reshape(16, G, NBUF, BS)
    dst = dst.reshape(16, G, NBUF, BS)
    return jnp.concatenate([src, dst], axis=2)


def kernel(x, edge_index_follows, edge_index_likes, edge_index_views,
           W_self, W_neigh, b, gamma, beta):
    eidx = [_prep_edge(e) for e in
            (edge_index_follows, edge_index_likes, edge_index_views)]
    zer = jnp.zeros((ROWS_PER_TILE, CW), jnp.float32)
    ones = jnp.ones((BS, CW), jnp.float32)

    degs = _sc_deg(eidx[0], eidx[1], eidx[2], ones, zer)

    xp = jnp.pad(x, ((0, N_PAD - N), (0, 0)))
    h_chunks = [xp[:, c * CW:(c + 1) * CW] for c in range(C)]

    for l in range(L):
        ag_a = _SC_AGG_A(h_chunks[0], h_chunks[1],
                         eidx[0], eidx[1], eidx[2], zer)
        ag_b = _SC_AGG_B(h_chunks[2], h_chunks[3],
                         eidx[0], eidx[1], eidx[2], zer)
        aggs = []
        for e in range(NE):
            aggs += [ag_a[e * 2], ag_a[e * 2 + 1],
                     ag_b[e * 2], ag_b[e * 2 + 1]]
        out, stats = _tc_conv(h_chunks, aggs, degs, W_self[l], W_neigh[l],
                              b[l], act=(l < L - 1))
        res = _tc_bn(out, stats, gamma[l][None, :], beta[l][None, :],
                     chunked=(l < L - 1))
        if l < L - 1:
            h_chunks = list(res)
        else:
            return res[0]


# speedup vs baseline: 1.9897x; 1.9897x over previous
import functools

import jax
import jax.numpy as jnp
from jax import lax
from jax.experimental import pallas as pl
from jax.experimental.pallas import tpu as pltpu
from jax.experimental.pallas import tpu_sc as plsc

N = 50000
D = 128
E = 200000
NE = 3
L = 3

HW = 64
N_PAD = 50048
ROWS_PER_TILE = N_PAD // 16

BS = 128
NBUF = 3
G = 34
NB = G * NBUF
E_PAD = 16 * NB * BS
PAD_IDX = N

_mesh = plsc.VectorSubcoreMesh(core_axis_name="c", subcore_axis_name="s")
_sc_params = pltpu.CompilerParams(use_tc_tiling_on_sc=False)


@functools.partial(
    pl.kernel,
    mesh=_mesh,
    compiler_params=_sc_params,
    out_type=[jax.ShapeDtypeStruct((2, N_PAD, 32), jnp.float32)
              for _ in range(NE)],
    scratch_types=[
        pltpu.VMEM((6, BS), jnp.int32),
        pltpu.VMEM((BS, 32), jnp.float32),
        pltpu.VMEM_SHARED((N_PAD, 32), jnp.float32),
    ],
)
def _sc_deg(i0, i1, i2, ones_h, zer_h, o0, o1, o2, idxv, onesv, acc):
    cid = lax.axis_index("c")
    sid = lax.axis_index("s")
    row_lo = sid * ROWS_PER_TILE
    pltpu.sync_copy(ones_h, onesv)
    for e, (iref, oref) in enumerate(((i0, o0), (i1, o1), (i2, o2))):
        pltpu.sync_copy(zer_h, acc.at[pl.ds(row_lo, ROWS_PER_TILE)])
        plsc.subcore_barrier()
        base_g = cid * (G // 2)

        def body(j, _, iref=iref):
            pltpu.sync_copy(iref.at[sid].at[base_g + j], idxv)
            for bb in range(NBUF):
                pltpu.sync_copy(onesv, acc.at[idxv.at[NBUF + bb]], add=True)
            return 0

        lax.fori_loop(0, G // 2, body, 0)
        plsc.subcore_barrier()
        for cval in range(2):
            @pl.when(cid == cval)
            def _(oref=oref, cval=cval):
                pltpu.sync_copy(acc.at[pl.ds(row_lo, ROWS_PER_TILE)],
                                oref.at[cval].at[pl.ds(row_lo, ROWS_PER_TILE)])
        plsc.subcore_barrier()


@functools.partial(
    pl.kernel,
    mesh=_mesh,
    compiler_params=_sc_params,
    out_type=[jax.ShapeDtypeStruct((N_PAD, HW), jnp.bfloat16)
              for _ in range(NE * 2)],
    scratch_types=[
        pltpu.VMEM((12, BS), jnp.int32),
        pltpu.VMEM((2, NBUF, BS, HW), jnp.bfloat16),
        pltpu.SemaphoreType.DMA,
        pltpu.SemaphoreType.DMA,
        pltpu.VMEM_SHARED((N_PAD, HW), jnp.bfloat16),
    ],
)
def _sc_agg(h0, h1, i0, i1, i2, zer_h, *rest):
    outs = rest[:NE * 2]
    idxv, rows, isem, gsem, acc = rest[NE * 2:]
    hrefs = (h0, h1)
    cid = lax.axis_index("c")
    sid = lax.axis_index("s")
    row_lo = sid * ROWS_PER_TILE

    def one_pass(href, oref, iref):
        tidx = iref.at[sid]
        pltpu.sync_copy(zer_h, acc.at[pl.ds(row_lo, ROWS_PER_TILE)])
        plsc.subcore_barrier()

        def fetch_idx(g, p):
            pltpu.async_copy(tidx.at[g], idxv.at[pl.ds(6 * p, 6)], isem)

        def wait_idx(p):
            pltpu.make_async_copy(tidx.at[0], idxv.at[pl.ds(6 * p, 6)],
                                  isem).wait()

        def gathers(p):
            for bb in range(NBUF):
                pltpu.async_copy(href.at[idxv.at[6 * p + bb]],
                                 rows.at[p].at[bb], gsem)

        def drain_scatter(p):
            for bb in range(NBUF):
                pltpu.make_async_copy(href.at[idxv.at[0]], rows.at[p].at[bb],
                                      gsem).wait()
                pltpu.sync_copy(rows.at[p].at[bb],
                                acc.at[idxv.at[6 * p + NBUF + bb]], add=True)

        fetch_idx(0, 0)
        fetch_idx(1, 1)
        wait_idx(0)
        gathers(0)

        def outer(o, _):
            for p in range(2):
                g = 2 * o + p
                drain_scatter(p)
                fetch_idx(g + 2, p)
                wait_idx(1 - p)
                gathers(1 - p)
            return 0

        lax.fori_loop(0, G // 2 - 1, outer, 0)
        drain_scatter(0)
        wait_idx(1)
        gathers(1)
        drain_scatter(1)

        plsc.subcore_barrier()
        pltpu.sync_copy(acc.at[pl.ds(row_lo, ROWS_PER_TILE)],
                        oref.at[pl.ds(row_lo, ROWS_PER_TILE)])
        plsc.subcore_barrier()

    for e, iref in enumerate((i0, i1, i2)):
        for cval in range(2):
            @pl.when(cid == cval)
            def _(cval=cval, e=e, iref=iref):
                one_pass(hrefs[cval], outs[e * 2 + cval], iref)


def _conv_body(h, a00, a01, a10, a11, a20, a21,
               g0, g1, g2, ws, wn, bs, out_ref, st_ref, sacc, *, act):
    i = pl.program_id(0)
    ab = ((a00, a01), (a10, a11), (a20, a21))
    db = (g0, g1, g2)
    hblk = h[...]
    acc = jnp.zeros((128, 128), jnp.float32)
    for e in range(NE):
        dg = db[e][...]
        deg = dg[0, :, 0:1] + dg[1, :, 0:1]
        inv = 1.0 / jnp.maximum(deg, 1.0)
        t = jnp.dot(hblk, ws[e], preferred_element_type=jnp.float32)
        for half in range(2):
            hn = ab[e][half][...].astype(jnp.float32) * inv
            t += jnp.dot(hn, wn[e, pl.ds(half * HW, HW), :],
                         preferred_element_type=jnp.float32)
        t += bs[pl.ds(e, 1), :]
        if act:
            t = jnp.maximum(t, 0.0)
        acc += t
    rows = i * 128 + lax.broadcasted_iota(jnp.int32, (128, 1), 0)
    acc = jnp.where(rows < N, acc, 0.0)
    out_ref[...] = acc
    st = jnp.concatenate(
        [jnp.sum(acc, axis=0, keepdims=True),
         jnp.sum(acc * acc, axis=0, keepdims=True)], axis=0)

    @pl.when(i == 0)
    def _():
        sacc[...] = st

    @pl.when(i > 0)
    def _():
        sacc[...] += st

    @pl.when(i == N_PAD // 128 - 1)
    def _():
        st_ref[...] = sacc[...]


def _tc_conv(h, aggs, degs, ws, wn, bias, act):
    nblk = N_PAD // 128
    aspec = pl.BlockSpec((128, HW), lambda i: (i, 0))
    dspec = pl.BlockSpec((2, 128, 32), lambda i: (0, i, 0))
    body = functools.partial(_conv_body, act=act)
    return pl.pallas_call(
        body,
        grid=(nblk,),
        in_specs=([pl.BlockSpec((128, 128), lambda i: (i, 0))]
                  + [aspec] * (NE * 2) + [dspec] * NE
                  + [pl.BlockSpec((NE, 128, 128), lambda i: (0, 0, 0)),
                     pl.BlockSpec((NE, 128, 128), lambda i: (0, 0, 0)),
                     pl.BlockSpec((NE, 128), lambda i: (0, 0))]),
        out_specs=[pl.BlockSpec((128, 128), lambda i: (i, 0)),
                   pl.BlockSpec((2, 128), lambda i: (0, 0))],
        out_shape=[jax.ShapeDtypeStruct((N_PAD, 128), jnp.float32),
                   jax.ShapeDtypeStruct((2, 128), jnp.float32)],
        scratch_shapes=[pltpu.VMEM((2, 128), jnp.float32)],
    )(h, *aggs, *degs, ws, wn, bias)


def _bn_body(x_ref, st_ref, g_ref, b_ref, *out_refs, chunked):
    i = pl.program_id(0)
    st = st_ref[...]
    mean = st[0:1, :] / N
    var = st[1:2, :] / N - mean * mean
    scale = g_ref[...] / jnp.sqrt(var + 1e-5)
    shift = b_ref[...] - mean * scale
    y = x_ref[...] * scale + shift
    rows = i * 128 + lax.broadcasted_iota(jnp.int32, (128, 1), 0)
    y = jnp.where(rows < N, y, 0.0)
    out_refs[0][...] = y
    if chunked:
        yh = y.astype(jnp.bfloat16)
        out_refs[1][...] = yh[:, :HW]
        out_refs[2][...] = yh[:, HW:]


def _tc_bn(out, stats, g, b, chunked):
    nblk = N_PAD // 128
    if chunked:
        out_specs = [pl.BlockSpec((128, 128), lambda i: (i, 0)),
                     pl.BlockSpec((128, HW), lambda i: (i, 0)),
                     pl.BlockSpec((128, HW), lambda i: (i, 0))]
        out_shape = [jax.ShapeDtypeStruct((N_PAD, 128), jnp.float32),
                     jax.ShapeDtypeStruct((N_PAD, HW), jnp.bfloat16),
                     jax.ShapeDtypeStruct((N_PAD, HW), jnp.bfloat16)]
    else:
        out_specs = [pl.BlockSpec((128, 128), lambda i: (i, 0))]
        out_shape = [jax.ShapeDtypeStruct((N, 128), jnp.float32)]
    body = functools.partial(_bn_body, chunked=chunked)
    return pl.pallas_call(
        body,
        grid=(nblk,),
        in_specs=[pl.BlockSpec((128, 128), lambda i: (i, 0)),
                  pl.BlockSpec((2, 128), lambda i: (0, 0)),
                  pl.BlockSpec((1, 128), lambda i: (0, 0)),
                  pl.BlockSpec((1, 128), lambda i: (0, 0))],
        out_specs=out_specs,
        out_shape=out_shape,
    )(out, stats, g, b)


def _prep_edge(ei):
    pad = E_PAD - E
    src = jnp.concatenate([ei[0], jnp.full((pad,), PAD_IDX, jnp.int32)])
    dst = jnp.concatenate([ei[1], jnp.full((pad,), PAD_IDX, jnp.int32)])
    src = src.reshape(16, G, NBUF, BS)
    dst = dst.reshape(16, G, NBUF, BS)
    return jnp.concatenate([src, dst], axis=2)


def kernel(x, edge_index_follows, edge_index_likes, edge_index_views,
           W_self, W_neigh, b, gamma, beta):
    eidx = [_prep_edge(e) for e in
            (edge_index_follows, edge_index_likes, edge_index_views)]
    zer32 = jnp.zeros((ROWS_PER_TILE, 32), jnp.float32)
    zer16 = jnp.zeros((ROWS_PER_TILE, HW), jnp.bfloat16)
    ones = jnp.ones((BS, 32), jnp.float32)

    degs = _sc_deg(eidx[0], eidx[1], eidx[2], ones, zer32)

    h = jnp.pad(x, ((0, N_PAD - N), (0, 0)))
    h16 = h.astype(jnp.bfloat16)
    halves = [h16[:, :HW], h16[:, HW:]]

    for l in range(L):
        aggs = _sc_agg(halves[0], halves[1],
                       eidx[0], eidx[1], eidx[2], zer16)
        out, stats = _tc_conv(h, aggs, degs, W_self[l], W_neigh[l],
                              b[l], act=(l < L - 1))
        res = _tc_bn(out, stats, gamma[l][None, :], beta[l][None, :],
                     chunked=(l < L - 1))
        if l < L - 1:
            h = res[0]
            halves = [res[1], res[2]]
        else:
            return res[0]


# speedup vs baseline: 2.0114x; 1.0109x over previous
import functools

import jax
import jax.numpy as jnp
from jax import lax
from jax.experimental import pallas as pl
from jax.experimental.pallas import tpu as pltpu
from jax.experimental.pallas import tpu_sc as plsc

N = 50000
D = 128
E = 200000
NE = 3
L = 3

HW = 64
N_PAD = 50048
ROWS_PER_TILE = N_PAD // 16

BS = 128
NBUF = 3
G = 34
NB = G * NBUF
E_PAD = 16 * NB * BS
PAD_IDX = N

_mesh = plsc.VectorSubcoreMesh(core_axis_name="c", subcore_axis_name="s")
_sc_params = pltpu.CompilerParams(use_tc_tiling_on_sc=False)


@functools.partial(
    pl.kernel,
    mesh=_mesh,
    compiler_params=_sc_params,
    out_type=[jax.ShapeDtypeStruct((2, N_PAD, 32), jnp.float32)
              for _ in range(NE)],
    scratch_types=[
        pltpu.VMEM((6, BS), jnp.int32),
        pltpu.VMEM((BS, 32), jnp.float32),
        pltpu.VMEM_SHARED((N_PAD, 32), jnp.float32),
    ],
)
def _sc_deg(i0, i1, i2, ones_h, zer_h, o0, o1, o2, idxv, onesv, acc):
    cid = lax.axis_index("c")
    sid = lax.axis_index("s")
    row_lo = sid * ROWS_PER_TILE
    pltpu.sync_copy(ones_h, onesv)
    for e, (iref, oref) in enumerate(((i0, o0), (i1, o1), (i2, o2))):
        pltpu.sync_copy(zer_h, acc.at[pl.ds(row_lo, ROWS_PER_TILE)])
        plsc.subcore_barrier()
        base_g = cid * (G // 2)

        def body(j, _, iref=iref):
            pltpu.sync_copy(iref.at[sid].at[base_g + j], idxv)
            for bb in range(NBUF):
                pltpu.sync_copy(onesv, acc.at[idxv.at[NBUF + bb]], add=True)
            return 0

        lax.fori_loop(0, G // 2, body, 0)
        plsc.subcore_barrier()
        for cval in range(2):
            @pl.when(cid == cval)
            def _(oref=oref, cval=cval):
                pltpu.sync_copy(acc.at[pl.ds(row_lo, ROWS_PER_TILE)],
                                oref.at[cval].at[pl.ds(row_lo, ROWS_PER_TILE)])
        plsc.subcore_barrier()


@functools.partial(
    pl.kernel,
    mesh=_mesh,
    compiler_params=_sc_params,
    out_type=[jax.ShapeDtypeStruct((N_PAD, HW), jnp.bfloat16)
              for _ in range(NE * 2)],
    scratch_types=[
        pltpu.VMEM((12, BS), jnp.int32),
        pltpu.VMEM((2, NBUF, BS), jnp.int32),
        pltpu.VMEM((2, NBUF, BS, HW), jnp.bfloat16),
        pltpu.SemaphoreType.DMA,
        pltpu.SemaphoreType.DMA,
        pltpu.SemaphoreType.DMA,
        pltpu.VMEM_SHARED((N_PAD, HW), jnp.bfloat16),
    ],
)
def _sc_agg(h0, h1, i0, i1, i2, zer_h, *rest):
    outs = rest[:NE * 2]
    idxv, sdst, rows, isem, gsem, ssem, acc = rest[NE * 2:]
    hrefs = (h0, h1)
    cid = lax.axis_index("c")
    sid = lax.axis_index("s")
    row_lo = sid * ROWS_PER_TILE

    def one_pass(href, oref, iref):
        tidx = iref.at[sid]
        pltpu.sync_copy(zer_h, acc.at[pl.ds(row_lo, ROWS_PER_TILE)])
        plsc.subcore_barrier()

        def fetch_idx(g, p):
            pltpu.async_copy(tidx.at[g], idxv.at[pl.ds(6 * p, 6)], isem)

        def wait_idx(p):
            pltpu.make_async_copy(tidx.at[0], idxv.at[pl.ds(6 * p, 6)],
                                  isem).wait()

        def gathers(p):
            for bb in range(NBUF):
                pltpu.async_copy(href.at[idxv.at[6 * p + bb]],
                                 rows.at[p].at[bb], gsem)

        def wait_gathers(p):
            for bb in range(NBUF):
                pltpu.make_async_copy(href.at[idxv.at[0]], rows.at[p].at[bb],
                                      gsem).wait()

        def start_scatters(p):
            for bb in range(NBUF):
                for j in range(BS // 16):
                    sdst[p, bb, pl.ds(j * 16, 16)] = (
                        idxv[6 * p + NBUF + bb, pl.ds(j * 16, 16)])
            for bb in range(NBUF):
                pltpu.async_copy(rows.at[p].at[bb],
                                 acc.at[sdst.at[p].at[bb]], ssem, add=True)

        def drain_scatters(p):
            for bb in range(NBUF):
                pltpu.make_async_copy(rows.at[p].at[bb],
                                      acc.at[sdst.at[p].at[bb]], ssem).wait()

        fetch_idx(0, 0)
        fetch_idx(1, 1)
        wait_idx(0)
        gathers(0)
        wait_gathers(0)
        start_scatters(0)
        fetch_idx(2, 0)
        wait_idx(1)
        gathers(1)

        def outer(o, _):
            for p in range(2):
                g = 2 * o + 1 + p
                q = (1 + p) % 2
                wait_gathers(q)
                start_scatters(q)
                fetch_idx(g + 2, q)
                wait_idx(1 - q)
                drain_scatters(1 - q)
                gathers(1 - q)
            return 0

        lax.fori_loop(0, (G - 4) // 2, outer, 0)
        wait_gathers(1)
        start_scatters(1)
        fetch_idx(G - 1, 1)
        wait_idx(0)
        drain_scatters(0)
        gathers(0)

        wait_gathers(0)
        start_scatters(0)
        wait_idx(1)
        drain_scatters(1)
        gathers(1)

        wait_gathers(1)
        start_scatters(1)
        drain_scatters(0)
        drain_scatters(1)

        plsc.subcore_barrier()
        pltpu.sync_copy(acc.at[pl.ds(row_lo, ROWS_PER_TILE)],
                        oref.at[pl.ds(row_lo, ROWS_PER_TILE)])
        plsc.subcore_barrier()

    for e, iref in enumerate((i0, i1, i2)):
        for cval in range(2):
            @pl.when(cid == cval)
            def _(cval=cval, e=e, iref=iref):
                one_pass(hrefs[cval], outs[e * 2 + cval], iref)


def _conv_body(h, a00, a01, a10, a11, a20, a21,
               g0, g1, g2, ws, wn, bs, out_ref, st_ref, sacc, *, act):
    i = pl.program_id(0)
    ab = ((a00, a01), (a10, a11), (a20, a21))
    db = (g0, g1, g2)
    hblk = h[...]
    acc = jnp.zeros((128, 128), jnp.float32)
    for e in range(NE):
        dg = db[e][...]
        deg = dg[0, :, 0:1] + dg[1, :, 0:1]
        inv = 1.0 / jnp.maximum(deg, 1.0)
        t = jnp.dot(hblk, ws[e], preferred_element_type=jnp.float32)
        for half in range(2):
            hn = ab[e][half][...].astype(jnp.float32) * inv
            t += jnp.dot(hn, wn[e, pl.ds(half * HW, HW), :],
                         preferred_element_type=jnp.float32)
        t += bs[pl.ds(e, 1), :]
        if act:
            t = jnp.maximum(t, 0.0)
        acc += t
    rows = i * 128 + lax.broadcasted_iota(jnp.int32, (128, 1), 0)
    acc = jnp.where(rows < N, acc, 0.0)
    out_ref[...] = acc
    st = jnp.concatenate(
        [jnp.sum(acc, axis=0, keepdims=True),
         jnp.sum(acc * acc, axis=0, keepdims=True)], axis=0)

    @pl.when(i == 0)
    def _():
        sacc[...] = st

    @pl.when(i > 0)
    def _():
        sacc[...] += st

    @pl.when(i == N_PAD // 128 - 1)
    def _():
        st_ref[...] = sacc[...]


def _tc_conv(h, aggs, degs, ws, wn, bias, act):
    nblk = N_PAD // 128
    aspec = pl.BlockSpec((128, HW), lambda i: (i, 0))
    dspec = pl.BlockSpec((2, 128, 32), lambda i: (0, i, 0))
    body = functools.partial(_conv_body, act=act)
    return pl.pallas_call(
        body,
        grid=(nblk,),
        in_specs=([pl.BlockSpec((128, 128), lambda i: (i, 0))]
                  + [aspec] * (NE * 2) + [dspec] * NE
                  + [pl.BlockSpec((NE, 128, 128), lambda i: (0, 0, 0)),
                     pl.BlockSpec((NE, 128, 128), lambda i: (0, 0, 0)),
                     pl.BlockSpec((NE, 128), lambda i: (0, 0))]),
        out_specs=[pl.BlockSpec((128, 128), lambda i: (i, 0)),
                   pl.BlockSpec((2, 128), lambda i: (0, 0))],
        out_shape=[jax.ShapeDtypeStruct((N_PAD, 128), jnp.float32),
                   jax.ShapeDtypeStruct((2, 128), jnp.float32)],
        scratch_shapes=[pltpu.VMEM((2, 128), jnp.float32)],
    )(h, *aggs, *degs, ws, wn, bias)


def _bn_body(x_ref, st_ref, g_ref, b_ref, *out_refs, chunked):
    i = pl.program_id(0)
    st = st_ref[...]
    mean = st[0:1, :] / N
    var = st[1:2, :] / N - mean * mean
    scale = g_ref[...] / jnp.sqrt(var + 1e-5)
    shift = b_ref[...] - mean * scale
    y = x_ref[...] * scale + shift
    rows = i * 128 + lax.broadcasted_iota(jnp.int32, (128, 1), 0)
    y = jnp.where(rows < N, y, 0.0)
    out_refs[0][...] = y
    if chunked:
        yh = y.astype(jnp.bfloat16)
        out_refs[1][...] = yh[:, :HW]
        out_refs[2][...] = yh[:, HW:]


def _tc_bn(out, stats, g, b, chunked):
    nblk = N_PAD // 128
    if chunked:
        out_specs = [pl.BlockSpec((128, 128), lambda i: (i, 0)),
                     pl.BlockSpec((128, HW), lambda i: (i, 0)),
                     pl.BlockSpec((128, HW), lambda i: (i, 0))]
        out_shape = [jax.ShapeDtypeStruct((N_PAD, 128), jnp.float32),
                     jax.ShapeDtypeStruct((N_PAD, HW), jnp.bfloat16),
                     jax.ShapeDtypeStruct((N_PAD, HW), jnp.bfloat16)]
    else:
        out_specs = [pl.BlockSpec((128, 128), lambda i: (i, 0))]
        out_shape = [jax.ShapeDtypeStruct((N, 128), jnp.float32)]
    body = functools.partial(_bn_body, chunked=chunked)
    return pl.pallas_call(
        body,
        grid=(nblk,),
        in_specs=[pl.BlockSpec((128, 128), lambda i: (i, 0)),
                  pl.BlockSpec((2, 128), lambda i: (0, 0)),
                  pl.BlockSpec((1, 128), lambda i: (0, 0)),
                  pl.BlockSpec((1, 128), lambda i: (0, 0))],
        out_specs=out_specs,
        out_shape=out_shape,
    )(out, stats, g, b)


def _prep_edge(ei):
    pad = E_PAD - E
    src = jnp.concatenate([ei[0], jnp.full((pad,), PAD_IDX, jnp.int32)])
    dst = jnp.concatenate([ei[1], jnp.full((pad,), PAD_IDX, jnp.int32)])
    src = src.reshape(16, G, NBUF, BS)
    dst = dst.reshape(16, G, NBUF, BS)
    return jnp.concatenate([src, dst], axis=2)


def kernel(x, edge_index_follows, edge_index_likes, edge_index_views,
           W_self, W_neigh, b, gamma, beta):
    eidx = [_prep_edge(e) for e in
            (edge_index_follows, edge_index_likes, edge_index_views)]
    zer32 = jnp.zeros((ROWS_PER_TILE, 32), jnp.float32)
    zer16 = jnp.zeros((ROWS_PER_TILE, HW), jnp.bfloat16)
    ones = jnp.ones((BS, 32), jnp.float32)

    degs = _sc_deg(eidx[0], eidx[1], eidx[2], ones, zer32)

    h = jnp.pad(x, ((0, N_PAD - N), (0, 0)))
    h16 = h.astype(jnp.bfloat16)
    halves = [h16[:, :HW], h16[:, HW:]]

    for l in range(L):
        aggs = _sc_agg(halves[0], halves[1],
                       eidx[0], eidx[1], eidx[2], zer16)
        out, stats = _tc_conv(h, aggs, degs, W_self[l], W_neigh[l],
                              b[l], act=(l < L - 1))
        res = _tc_bn(out, stats, gamma[l][None, :], beta[l][None, :],
                     chunked=(l < L - 1))
        if l < L - 1:
            h = res[0]
            halves = [res[1], res[2]]
        else:
            return res[0]


# speedup vs baseline: 2.0707x; 1.0295x over previous
import functools

import jax
import jax.numpy as jnp
from jax import lax
from jax.experimental import pallas as pl
from jax.experimental.pallas import tpu as pltpu
from jax.experimental.pallas import tpu_sc as plsc

N = 50000
D = 128
E = 200000
NE = 3
L = 3

HW = 64
N_PAD = 50048
ROWS_PER_TILE = N_PAD // 16

BS = 128
NBUF = 3
G = 34
NB = G * NBUF
E_PAD = 16 * NB * BS
PAD_IDX = N

_mesh = plsc.VectorSubcoreMesh(core_axis_name="c", subcore_axis_name="s")
_sc_params = pltpu.CompilerParams(use_tc_tiling_on_sc=False)


@functools.partial(
    pl.kernel,
    mesh=_mesh,
    compiler_params=_sc_params,
    out_type=[jax.ShapeDtypeStruct((2, N_PAD, 32), jnp.float32)
              for _ in range(NE)],
    scratch_types=[
        pltpu.VMEM((6, BS), jnp.int32),
        pltpu.VMEM((BS, 32), jnp.float32),
        pltpu.VMEM_SHARED((N_PAD, 32), jnp.float32),
    ],
)
def _sc_deg(i0, i1, i2, ones_h, zer_h, o0, o1, o2, idxv, onesv, acc):
    cid = lax.axis_index("c")
    sid = lax.axis_index("s")
    row_lo = sid * ROWS_PER_TILE
    pltpu.sync_copy(ones_h, onesv)
    for e, (iref, oref) in enumerate(((i0, o0), (i1, o1), (i2, o2))):
        pltpu.sync_copy(zer_h, acc.at[pl.ds(row_lo, ROWS_PER_TILE)])
        plsc.subcore_barrier()
        base_g = cid * (G // 2)

        def body(j, _, iref=iref):
            pltpu.sync_copy(iref.at[sid].at[base_g + j], idxv)
            for bb in range(NBUF):
                pltpu.sync_copy(onesv, acc.at[idxv.at[NBUF + bb]], add=True)
            return 0

        lax.fori_loop(0, G // 2, body, 0)
        plsc.subcore_barrier()
        for cval in range(2):
            @pl.when(cid == cval)
            def _(oref=oref, cval=cval):
                pltpu.sync_copy(acc.at[pl.ds(row_lo, ROWS_PER_TILE)],
                                oref.at[cval].at[pl.ds(row_lo, ROWS_PER_TILE)])
        plsc.subcore_barrier()


@functools.partial(
    pl.kernel,
    mesh=_mesh,
    compiler_params=_sc_params,
    out_type=[jax.ShapeDtypeStruct((N_PAD, HW), jnp.bfloat16)
              for _ in range(NE * 2)],
    scratch_types=[
        pltpu.VMEM((12, BS), jnp.int32),
        pltpu.VMEM((2, NBUF, BS), jnp.int32),
        pltpu.VMEM((2, NBUF, BS, HW), jnp.bfloat16),
        pltpu.SemaphoreType.DMA,
        pltpu.SemaphoreType.DMA,
        pltpu.SemaphoreType.DMA,
        pltpu.VMEM_SHARED((N_PAD, HW), jnp.bfloat16),
    ],
)
def _sc_agg(h0, h1, i0, i1, i2, zer_h, *rest):
    outs = rest[:NE * 2]
    idxv, sdst, rows, isem, gsem, ssem, acc = rest[NE * 2:]
    hrefs = (h0, h1)
    cid = lax.axis_index("c")
    sid = lax.axis_index("s")
    row_lo = sid * ROWS_PER_TILE

    def one_pass(href, oref, iref):
        tidx = iref.at[sid]
        pltpu.sync_copy(zer_h, acc.at[pl.ds(row_lo, ROWS_PER_TILE)])
        plsc.subcore_barrier()

        def fetch_idx(g, p):
            pltpu.async_copy(tidx.at[g], idxv.at[pl.ds(6 * p, 6)], isem)

        def wait_idx(p):
            pltpu.make_async_copy(tidx.at[0], idxv.at[pl.ds(6 * p, 6)],
                                  isem).wait()

        def gathers(p):
            for bb in range(NBUF):
                pltpu.async_copy(href.at[idxv.at[6 * p + bb]],
                                 rows.at[p].at[bb], gsem)

        def wait_gathers(p):
            for bb in range(NBUF):
                pltpu.make_async_copy(href.at[idxv.at[0]], rows.at[p].at[bb],
                                      gsem).wait()

        def start_scatters(p):
            for bb in range(NBUF):
                for j in range(BS // 16):
                    sdst[p, bb, pl.ds(j * 16, 16)] = (
                        idxv[6 * p + NBUF + bb, pl.ds(j * 16, 16)])
            for bb in range(NBUF):
                pltpu.async_copy(rows.at[p].at[bb],
                                 acc.at[sdst.at[p].at[bb]], ssem, add=True)

        def drain_scatters(p):
            for bb in range(NBUF):
                pltpu.make_async_copy(rows.at[p].at[bb],
                                      acc.at[sdst.at[p].at[bb]], ssem).wait()

        fetch_idx(0, 0)
        fetch_idx(1, 1)
        wait_idx(0)
        gathers(0)
        wait_idx(1)
        gathers(1)
        wait_gathers(0)
        start_scatters(0)
        fetch_idx(2, 0)

        def outer(o, _):
            for p in range(2):
                g = 2 * o + 1 + p
                q = (1 + p) % 2
                wait_idx(1 - q)
                drain_scatters(1 - q)
                gathers(1 - q)
                wait_gathers(q)
                start_scatters(q)
                fetch_idx(g + 2, q)
            return 0

        lax.fori_loop(0, (G - 4) // 2, outer, 0)
        wait_idx(0)
        drain_scatters(0)
        gathers(0)
        wait_gathers(1)
        start_scatters(1)
        fetch_idx(G - 1, 1)

        wait_idx(1)
        drain_scatters(1)
        gathers(1)
        wait_gathers(0)
        start_scatters(0)

        drain_scatters(0)
        wait_gathers(1)
        start_scatters(1)
        drain_scatters(1)

        plsc.subcore_barrier()
        pltpu.sync_copy(acc.at[pl.ds(row_lo, ROWS_PER_TILE)],
                        oref.at[pl.ds(row_lo, ROWS_PER_TILE)])
        plsc.subcore_barrier()

    for e, iref in enumerate((i0, i1, i2)):
        for cval in range(2):
            @pl.when(cid == cval)
            def _(cval=cval, e=e, iref=iref):
                one_pass(hrefs[cval], outs[e * 2 + cval], iref)


def _conv_body(h, a00, a01, a10, a11, a20, a21,
               g0, g1, g2, ws, wn, bs, out_ref, st_ref, sacc, *, act):
    i = pl.program_id(0)
    ab = ((a00, a01), (a10, a11), (a20, a21))
    db = (g0, g1, g2)
    hblk = h[...]
    acc = jnp.zeros((128, 128), jnp.float32)
    for e in range(NE):
        dg = db[e][...]
        deg = dg[0, :, 0:1] + dg[1, :, 0:1]
        inv = 1.0 / jnp.maximum(deg, 1.0)
        t = jnp.dot(hblk, ws[e], preferred_element_type=jnp.float32)
        for half in range(2):
            hn = ab[e][half][...].astype(jnp.float32) * inv
            t += jnp.dot(hn, wn[e, pl.ds(half * HW, HW), :],
                         preferred_element_type=jnp.float32)
        t += bs[pl.ds(e, 1), :]
        if act:
            t = jnp.maximum(t, 0.0)
        acc += t
    rows = i * 128 + lax.broadcasted_iota(jnp.int32, (128, 1), 0)
    acc = jnp.where(rows < N, acc, 0.0)
    out_ref[...] = acc
    st = jnp.concatenate(
        [jnp.sum(acc, axis=0, keepdims=True),
         jnp.sum(acc * acc, axis=0, keepdims=True)], axis=0)

    @pl.when(i == 0)
    def _():
        sacc[...] = st

    @pl.when(i > 0)
    def _():
        sacc[...] += st

    @pl.when(i == N_PAD // 128 - 1)
    def _():
        st_ref[...] = sacc[...]


def _tc_conv(h, aggs, degs, ws, wn, bias, act):
    nblk = N_PAD // 128
    aspec = pl.BlockSpec((128, HW), lambda i: (i, 0))
    dspec = pl.BlockSpec((2, 128, 32), lambda i: (0, i, 0))
    body = functools.partial(_conv_body, act=act)
    return pl.pallas_call(
        body,
        grid=(nblk,),
        in_specs=([pl.BlockSpec((128, 128), lambda i: (i, 0))]
                  + [aspec] * (NE * 2) + [dspec] * NE
                  + [pl.BlockSpec((NE, 128, 128), lambda i: (0, 0, 0)),
                     pl.BlockSpec((NE, 128, 128), lambda i: (0, 0, 0)),
                     pl.BlockSpec((NE, 128), lambda i: (0, 0))]),
        out_specs=[pl.BlockSpec((128, 128), lambda i: (i, 0)),
                   pl.BlockSpec((2, 128), lambda i: (0, 0))],
        out_shape=[jax.ShapeDtypeStruct((N_PAD, 128), jnp.float32),
                   jax.ShapeDtypeStruct((2, 128), jnp.float32)],
        scratch_shapes=[pltpu.VMEM((2, 128), jnp.float32)],
    )(h, *aggs, *degs, ws, wn, bias)


def _bn_body(x_ref, st_ref, g_ref, b_ref, *out_refs, chunked):
    i = pl.program_id(0)
    st = st_ref[...]
    mean = st[0:1, :] / N
    var = st[1:2, :] / N - mean * mean
    scale = g_ref[...] / jnp.sqrt(var + 1e-5)
    shift = b_ref[...] - mean * scale
    y = x_ref[...] * scale + shift
    rows = i * 128 + lax.broadcasted_iota(jnp.int32, (128, 1), 0)
    y = jnp.where(rows < N, y, 0.0)
    out_refs[0][...] = y
    if chunked:
        yh = y.astype(jnp.bfloat16)
        out_refs[1][...] = yh[:, :HW]
        out_refs[2][...] = yh[:, HW:]


def _tc_bn(out, stats, g, b, chunked):
    nblk = N_PAD // 128
    if chunked:
        out_specs = [pl.BlockSpec((128, 128), lambda i: (i, 0)),
                     pl.BlockSpec((128, HW), lambda i: (i, 0)),
                     pl.BlockSpec((128, HW), lambda i: (i, 0))]
        out_shape = [jax.ShapeDtypeStruct((N_PAD, 128), jnp.float32),
                     jax.ShapeDtypeStruct((N_PAD, HW), jnp.bfloat16),
                     jax.ShapeDtypeStruct((N_PAD, HW), jnp.bfloat16)]
    else:
        out_specs = [pl.BlockSpec((128, 128), lambda i: (i, 0))]
        out_shape = [jax.ShapeDtypeStruct((N, 128), jnp.float32)]
    body = functools.partial(_bn_body, chunked=chunked)
    return pl.pallas_call(
        body,
        grid=(nblk,),
        in_specs=[pl.BlockSpec((128, 128), lambda i: (i, 0)),
                  pl.BlockSpec((2, 128), lambda i: (0, 0)),
                  pl.BlockSpec((1, 128), lambda i: (0, 0)),
                  pl.BlockSpec((1, 128), lambda i: (0, 0))],
        out_specs=out_specs,
        out_shape=out_shape,
    )(out, stats, g, b)


def _prep_edge(ei):
    pad = E_PAD - E
    src = jnp.concatenate([ei[0], jnp.full((pad,), PAD_IDX, jnp.int32)])
    dst = jnp.concatenate([ei[1], jnp.full((pad,), PAD_IDX, jnp.int32)])
    src = src.reshape(16, G, NBUF, BS)
    dst = dst.reshape(16, G, NBUF, BS)
    return jnp.concatenate([src, dst], axis=2)


def kernel(x, edge_index_follows, edge_index_likes, edge_index_views,
           W_self, W_neigh, b, gamma, beta):
    eidx = [_prep_edge(e) for e in
            (edge_index_follows, edge_index_likes, edge_index_views)]
    zer32 = jnp.zeros((ROWS_PER_TILE, 32), jnp.float32)
    zer16 = jnp.zeros((ROWS_PER_TILE, HW), jnp.bfloat16)
    ones = jnp.ones((BS, 32), jnp.float32)

    degs = _sc_deg(eidx[0], eidx[1], eidx[2], ones, zer32)

    h = jnp.pad(x, ((0, N_PAD - N), (0, 0)))
    h16 = h.astype(jnp.bfloat16)
    halves = [h16[:, :HW], h16[:, HW:]]

    for l in range(L):
        aggs = _sc_agg(halves[0], halves[1],
                       eidx[0], eidx[1], eidx[2], zer16)
        out, stats = _tc_conv(h, aggs, degs, W_self[l], W_neigh[l],
                              b[l], act=(l < L - 1))
        res = _tc_bn(out, stats, gamma[l][None, :], beta[l][None, :],
                     chunked=(l < L - 1))
        if l < L - 1:
            h = res[0]
            halves = [res[1], res[2]]
        else:
            return res[0]


# speedup vs baseline: 2.4602x; 1.1881x over previous
import functools

import jax
import jax.numpy as jnp
from jax import lax
from jax.experimental import pallas as pl
from jax.experimental.pallas import tpu as pltpu
from jax.experimental.pallas import tpu_sc as plsc

N = 50000
D = 128
E = 200000
NE = 3
L = 3

HW = 64
N_PAD = 50176
ROWS_PER_TILE = N_PAD // 16
TBLK = 256

BS = 128
NBUF = 3
G = 34
NB = G * NBUF
E_PAD = 16 * NB * BS
PAD_IDX = N

_mesh = plsc.VectorSubcoreMesh(core_axis_name="c", subcore_axis_name="s")
_sc_params = pltpu.CompilerParams(use_tc_tiling_on_sc=False)


@functools.partial(
    pl.kernel,
    mesh=_mesh,
    compiler_params=_sc_params,
    out_type=[jax.ShapeDtypeStruct((2, N_PAD, 32), jnp.float32)
              for _ in range(NE)],
    scratch_types=[
        pltpu.VMEM((6, BS), jnp.int32),
        pltpu.VMEM((BS, 32), jnp.float32),
        pltpu.VMEM_SHARED((N_PAD, 32), jnp.float32),
    ],
)
def _sc_deg(i0, i1, i2, ones_h, zer_h, o0, o1, o2, idxv, onesv, acc):
    cid = lax.axis_index("c")
    sid = lax.axis_index("s")
    row_lo = sid * ROWS_PER_TILE
    pltpu.sync_copy(ones_h, onesv)
    for e, (iref, oref) in enumerate(((i0, o0), (i1, o1), (i2, o2))):
        pltpu.sync_copy(zer_h, acc.at[pl.ds(row_lo, ROWS_PER_TILE)])
        plsc.subcore_barrier()
        base_g = cid * (G // 2)

        def body(j, _, iref=iref):
            pltpu.sync_copy(iref.at[sid].at[base_g + j], idxv)
            for bb in range(NBUF):
                pltpu.sync_copy(onesv, acc.at[idxv.at[NBUF + bb]], add=True)
            return 0

        lax.fori_loop(0, G // 2, body, 0)
        plsc.subcore_barrier()
        for cval in range(2):
            @pl.when(cid == cval)
            def _(oref=oref, cval=cval):
                pltpu.sync_copy(acc.at[pl.ds(row_lo, ROWS_PER_TILE)],
                                oref.at[cval].at[pl.ds(row_lo, ROWS_PER_TILE)])
        plsc.subcore_barrier()


@functools.partial(
    pl.kernel,
    mesh=_mesh,
    compiler_params=_sc_params,
    out_type=[jax.ShapeDtypeStruct((N_PAD, HW), jnp.bfloat16)
              for _ in range(NE * 2)],
    scratch_types=[
        pltpu.VMEM((12, BS), jnp.int32),
        pltpu.VMEM((2, NBUF, BS), jnp.int32),
        pltpu.VMEM((2, NBUF, BS, HW), jnp.bfloat16),
        pltpu.SemaphoreType.DMA,
        pltpu.SemaphoreType.DMA,
        pltpu.SemaphoreType.DMA,
        pltpu.VMEM_SHARED((N_PAD, HW), jnp.bfloat16),
    ],
)
def _sc_agg(h0, h1, i0, i1, i2, zer_h, *rest):
    outs = rest[:NE * 2]
    idxv, sdst, rows, isem, gsem, ssem, acc = rest[NE * 2:]
    hrefs = (h0, h1)
    cid = lax.axis_index("c")
    sid = lax.axis_index("s")
    row_lo = sid * ROWS_PER_TILE

    def one_pass(href, oref, iref):
        tidx = iref.at[sid]
        pltpu.sync_copy(zer_h, acc.at[pl.ds(row_lo, ROWS_PER_TILE)])
        plsc.subcore_barrier()

        def fetch_idx(g, p):
            pltpu.async_copy(tidx.at[g], idxv.at[pl.ds(6 * p, 6)], isem)

        def wait_idx(p):
            pltpu.make_async_copy(tidx.at[0], idxv.at[pl.ds(6 * p, 6)],
                                  isem).wait()

        def gathers(p):
            for bb in range(NBUF):
                pltpu.async_copy(href.at[idxv.at[6 * p + bb]],
                                 rows.at[p].at[bb], gsem)

        def wait_gathers(p):
            for bb in range(NBUF):
                pltpu.make_async_copy(href.at[idxv.at[0]], rows.at[p].at[bb],
                                      gsem).wait()

        def start_scatters(p):
            for bb in range(NBUF):
                for j in range(BS // 16):
                    sdst[p, bb, pl.ds(j * 16, 16)] = (
                        idxv[6 * p + NBUF + bb, pl.ds(j * 16, 16)])
            for bb in range(NBUF):
                pltpu.async_copy(rows.at[p].at[bb],
                                 acc.at[sdst.at[p].at[bb]], ssem, add=True)

        def drain_scatters(p):
            for bb in range(NBUF):
                pltpu.make_async_copy(rows.at[p].at[bb],
                                      acc.at[sdst.at[p].at[bb]], ssem).wait()

        fetch_idx(0, 0)
        fetch_idx(1, 1)
        wait_idx(0)
        gathers(0)
        wait_idx(1)
        gathers(1)
        wait_gathers(0)
        start_scatters(0)
        fetch_idx(2, 0)

        def outer(o, _):
            for p in range(2):
                g = 2 * o + 1 + p
                q = (1 + p) % 2
                wait_idx(1 - q)
                drain_scatters(1 - q)
                gathers(1 - q)
                wait_gathers(q)
                start_scatters(q)
                fetch_idx(g + 2, q)
            return 0

        lax.fori_loop(0, (G - 4) // 2, outer, 0)
        wait_idx(0)
        drain_scatters(0)
        gathers(0)
        wait_gathers(1)
        start_scatters(1)
        fetch_idx(G - 1, 1)

        wait_idx(1)
        drain_scatters(1)
        gathers(1)
        wait_gathers(0)
        start_scatters(0)

        drain_scatters(0)
        wait_gathers(1)
        start_scatters(1)
        drain_scatters(1)

        plsc.subcore_barrier()
        pltpu.sync_copy(acc.at[pl.ds(row_lo, ROWS_PER_TILE)],
                        oref.at[pl.ds(row_lo, ROWS_PER_TILE)])
        plsc.subcore_barrier()

    for e, iref in enumerate((i0, i1, i2)):
        for cval in range(2):
            @pl.when(cid == cval)
            def _(cval=cval, e=e, iref=iref):
                one_pass(hrefs[cval], outs[e * 2 + cval], iref)


def _self_body(h, ws, s0, s1, s2):
    hblk = h[...]
    for e, sref in enumerate((s0, s1, s2)):
        sref[...] = jnp.dot(hblk, ws[e], preferred_element_type=jnp.float32)


def _tc_self(h, ws):
    nblk = N_PAD // TBLK
    return pl.pallas_call(
        _self_body,
        grid=(nblk,),
        in_specs=[pl.BlockSpec((TBLK, 128), lambda i: (i, 0)),
                  pl.BlockSpec((NE, 128, 128), lambda i: (0, 0, 0))],
        out_specs=[pl.BlockSpec((TBLK, 128), lambda i: (i, 0))
                   for _ in range(NE)],
        out_shape=[jax.ShapeDtypeStruct((N_PAD, 128), jnp.float32)
                   for _ in range(NE)],
    )(h, ws)


def _conv_body(s0, s1, s2, a00, a01, a10, a11, a20, a21,
               g0, g1, g2, wn, bs, out_ref, st_ref, sacc, *, act):
    i = pl.program_id(0)
    sb = (s0, s1, s2)
    ab = ((a00, a01), (a10, a11), (a20, a21))
    db = (g0, g1, g2)
    acc = jnp.zeros((TBLK, 128), jnp.float32)
    for e in range(NE):
        dg = db[e][...]
        deg = dg[0, :, 0:1] + dg[1, :, 0:1]
        inv = 1.0 / jnp.maximum(deg, 1.0)
        t = sb[e][...]
        for half in range(2):
            hn = ab[e][half][...].astype(jnp.float32) * inv
            t += jnp.dot(hn, wn[e, pl.ds(half * HW, HW), :],
                         preferred_element_type=jnp.float32)
        t += bs[pl.ds(e, 1), :]
        if act:
            t = jnp.maximum(t, 0.0)
        acc += t
    rows = i * TBLK + lax.broadcasted_iota(jnp.int32, (TBLK, 1), 0)
    acc = jnp.where(rows < N, acc, 0.0)
    out_ref[...] = acc
    st = jnp.concatenate(
        [jnp.sum(acc, axis=0, keepdims=True),
         jnp.sum(acc * acc, axis=0, keepdims=True)], axis=0)

    @pl.when(i == 0)
    def _():
        sacc[...] = st

    @pl.when(i > 0)
    def _():
        sacc[...] += st

    @pl.when(i == N_PAD // TBLK - 1)
    def _():
        st_ref[...] = sacc[...]


def _tc_conv(selfs, aggs, degs, wn, bias, act):
    nblk = N_PAD // TBLK
    sspec = pl.BlockSpec((TBLK, 128), lambda i: (i, 0))
    aspec = pl.BlockSpec((TBLK, HW), lambda i: (i, 0))
    dspec = pl.BlockSpec((2, TBLK, 32), lambda i: (0, i, 0))
    body = functools.partial(_conv_body, act=act)
    return pl.pallas_call(
        body,
        grid=(nblk,),
        in_specs=([sspec] * NE + [aspec] * (NE * 2) + [dspec] * NE
                  + [pl.BlockSpec((NE, 128, 128), lambda i: (0, 0, 0)),
                     pl.BlockSpec((NE, 128), lambda i: (0, 0))]),
        out_specs=[pl.BlockSpec((TBLK, 128), lambda i: (i, 0)),
                   pl.BlockSpec((2, 128), lambda i: (0, 0))],
        out_shape=[jax.ShapeDtypeStruct((N_PAD, 128), jnp.float32),
                   jax.ShapeDtypeStruct((2, 128), jnp.float32)],
        scratch_shapes=[pltpu.VMEM((2, 128), jnp.float32)],
    )(*selfs, *aggs, *degs, wn, bias)


def _bn_body(x_ref, st_ref, g_ref, b_ref, *out_refs, chunked):
    i = pl.program_id(0)
    st = st_ref[...]
    mean = st[0:1, :] / N
    var = st[1:2, :] / N - mean * mean
    scale = g_ref[...] / jnp.sqrt(var + 1e-5)
    shift = b_ref[...] - mean * scale
    y = x_ref[...] * scale + shift
    rows = i * TBLK + lax.broadcasted_iota(jnp.int32, (TBLK, 1), 0)
    y = jnp.where(rows < N, y, 0.0)
    out_refs[0][...] = y
    if chunked:
        yh = y.astype(jnp.bfloat16)
        out_refs[1][...] = yh[:, :HW]
        out_refs[2][...] = yh[:, HW:]


def _tc_bn(out, stats, g, b, chunked):
    nblk = N_PAD // TBLK
    if chunked:
        out_specs = [pl.BlockSpec((TBLK, 128), lambda i: (i, 0)),
                     pl.BlockSpec((TBLK, HW), lambda i: (i, 0)),
                     pl.BlockSpec((TBLK, HW), lambda i: (i, 0))]
        out_shape = [jax.ShapeDtypeStruct((N_PAD, 128), jnp.float32),
                     jax.ShapeDtypeStruct((N_PAD, HW), jnp.bfloat16),
                     jax.ShapeDtypeStruct((N_PAD, HW), jnp.bfloat16)]
    else:
        out_specs = [pl.BlockSpec((TBLK, 128), lambda i: (i, 0))]
        out_shape = [jax.ShapeDtypeStruct((N, 128), jnp.float32)]
    body = functools.partial(_bn_body, chunked=chunked)
    return pl.pallas_call(
        body,
        grid=(nblk,),
        in_specs=[pl.BlockSpec((TBLK, 128), lambda i: (i, 0)),
                  pl.BlockSpec((2, 128), lambda i: (0, 0)),
                  pl.BlockSpec((1, 128), lambda i: (0, 0)),
                  pl.BlockSpec((1, 128), lambda i: (0, 0))],
        out_specs=out_specs,
        out_shape=out_shape,
    )(out, stats, g, b)


def _prep_edge(ei):
    pad = E_PAD - E
    src = jnp.concatenate([ei[0], jnp.full((pad,), PAD_IDX, jnp.int32)])
    dst = jnp.concatenate([ei[1], jnp.full((pad,), PAD_IDX, jnp.int32)])
    src = src.reshape(16, G, NBUF, BS)
    dst = dst.reshape(16, G, NBUF, BS)
    return jnp.concatenate([src, dst], axis=2)


def kernel(x, edge_index_follows, edge_index_likes, edge_index_views,
           W_self, W_neigh, b, gamma, beta):
    eidx = [_prep_edge(e) for e in
            (edge_index_follows, edge_index_likes, edge_index_views)]
    zer32 = jnp.zeros((ROWS_PER_TILE, 32), jnp.float32)
    zer16 = jnp.zeros((ROWS_PER_TILE, HW), jnp.bfloat16)
    ones = jnp.ones((BS, 32), jnp.float32)

    degs = _sc_deg(eidx[0], eidx[1], eidx[2], ones, zer32)

    h = jnp.pad(x, ((0, N_PAD - N), (0, 0)))
    h16 = h.astype(jnp.bfloat16)
    halves = [h16[:, :HW], h16[:, HW:]]

    for l in range(L):
        aggs = _sc_agg(halves[0], halves[1],
                       eidx[0], eidx[1], eidx[2], zer16)
        selfs = _tc_self(h, W_self[l])
        out, stats = _tc_conv(selfs, aggs, degs, W_neigh[l],
                              b[l], act=(l < L - 1))
        res = _tc_bn(out, stats, gamma[l][None, :], beta[l][None, :],
                     chunked=(l < L - 1))
        if l < L - 1:
            h = res[0]
            halves = [res[1], res[2]]
        else:
            return res[0]


# speedup vs baseline: 2.4709x; 1.0044x over previous
import functools

import jax
import jax.numpy as jnp
from jax import lax
from jax.experimental import pallas as pl
from jax.experimental.pallas import tpu as pltpu
from jax.experimental.pallas import tpu_sc as plsc

N = 50000
D = 128
E = 200000
NE = 3
L = 3

HW = 64
N_PAD = 50176
ROWS_PER_TILE = N_PAD // 16
TBLK = 256

BS = 128
NBUF = 3
G = 34
NB = G * NBUF
E_PAD = 16 * NB * BS
PAD_IDX = N

_mesh = plsc.VectorSubcoreMesh(core_axis_name="c", subcore_axis_name="s")
_sc_params = pltpu.CompilerParams(use_tc_tiling_on_sc=False)


@functools.partial(
    pl.kernel,
    mesh=_mesh,
    compiler_params=_sc_params,
    out_type=[jax.ShapeDtypeStruct((2, N_PAD, 32), jnp.float32)
              for _ in range(NE)],
    scratch_types=[
        pltpu.VMEM((12, BS), jnp.int32),
        pltpu.VMEM((2, NBUF, BS), jnp.int32),
        pltpu.VMEM((BS, 32), jnp.float32),
        pltpu.SemaphoreType.DMA,
        pltpu.SemaphoreType.DMA,
        pltpu.VMEM_SHARED((N_PAD, 32), jnp.float32),
    ],
)
def _sc_deg(i0, i1, i2, ones_h, zer_h, o0, o1, o2,
            idxv, sdst, onesv, isem, ssem, acc):
    cid = lax.axis_index("c")
    sid = lax.axis_index("s")
    row_lo = sid * ROWS_PER_TILE
    pltpu.sync_copy(ones_h, onesv)
    GH = G // 2
    for e, (iref, oref) in enumerate(((i0, o0), (i1, o1), (i2, o2))):
        tidx = iref.at[sid]
        pltpu.sync_copy(zer_h, acc.at[pl.ds(row_lo, ROWS_PER_TILE)])
        plsc.subcore_barrier()
        base_g = cid * GH

        def fetch_idx(j, p, tidx=tidx):
            pltpu.async_copy(tidx.at[base_g + j], idxv.at[pl.ds(6 * p, 6)],
                             isem)

        def wait_idx(p, tidx=tidx):
            pltpu.make_async_copy(tidx.at[0], idxv.at[pl.ds(6 * p, 6)],
                                  isem).wait()

        def start_scatters(p):
            for bb in range(NBUF):
                for jj in range(BS // 16):
                    sdst[p, bb, pl.ds(jj * 16, 16)] = (
                        idxv[6 * p + NBUF + bb, pl.ds(jj * 16, 16)])
            for bb in range(NBUF):
                pltpu.async_copy(onesv, acc.at[sdst.at[p].at[bb]], ssem,
                                 add=True)

        def drain_scatters(p):
            for bb in range(NBUF):
                pltpu.make_async_copy(onesv, acc.at[sdst.at[p].at[bb]],
                                      ssem).wait()

        fetch_idx(0, 0)
        fetch_idx(1, 1)
        wait_idx(0)
        start_scatters(0)
        fetch_idx(2, 0)

        def outer(o, _):
            for p in range(2):
                j = 2 * o + 1 + p
                q = (1 + p) % 2
                wait_idx(q)
                start_scatters(q)
                drain_scatters(1 - q)
                fetch_idx(j + 2, q)
            return 0

        lax.fori_loop(0, (GH - 3) // 2, outer, 0)
        wait_idx(1)
        start_scatters(1)
        drain_scatters(0)
        wait_idx(0)
        start_scatters(0)
        drain_scatters(1)
        drain_scatters(0)
        plsc.subcore_barrier()
        for cval in range(2):
            @pl.when(cid == cval)
            def _(oref=oref, cval=cval):
                pltpu.sync_copy(acc.at[pl.ds(row_lo, ROWS_PER_TILE)],
                                oref.at[cval].at[pl.ds(row_lo, ROWS_PER_TILE)])
        plsc.subcore_barrier()


@functools.partial(
    pl.kernel,
    mesh=_mesh,
    compiler_params=_sc_params,
    out_type=[jax.ShapeDtypeStruct((N_PAD, HW), jnp.bfloat16)
              for _ in range(NE * 2)],
    scratch_types=[
        pltpu.VMEM((12, BS), jnp.int32),
        pltpu.VMEM((2, NBUF, BS), jnp.int32),
        pltpu.VMEM((2, NBUF, BS, HW), jnp.bfloat16),
        pltpu.SemaphoreType.DMA,
        pltpu.SemaphoreType.DMA,
        pltpu.SemaphoreType.DMA,
        pltpu.VMEM_SHARED((N_PAD, HW), jnp.bfloat16),
    ],
)
def _sc_agg(h0, h1, i0, i1, i2, zer_h, *rest):
    outs = rest[:NE * 2]
    idxv, sdst, rows, isem, gsem, ssem, acc = rest[NE * 2:]
    hrefs = (h0, h1)
    cid = lax.axis_index("c")
    sid = lax.axis_index("s")
    row_lo = sid * ROWS_PER_TILE

    def one_pass(href, oref, iref):
        tidx = iref.at[sid]
        pltpu.sync_copy(zer_h, acc.at[pl.ds(row_lo, ROWS_PER_TILE)])
        plsc.subcore_barrier()

        def fetch_idx(g, p):
            pltpu.async_copy(tidx.at[g], idxv.at[pl.ds(6 * p, 6)], isem)

        def wait_idx(p):
            pltpu.make_async_copy(tidx.at[0], idxv.at[pl.ds(6 * p, 6)],
                                  isem).wait()

        def gathers(p):
            for bb in range(NBUF):
                pltpu.async_copy(href.at[idxv.at[6 * p + bb]],
                                 rows.at[p].at[bb], gsem)

        def wait_gathers(p):
            for bb in range(NBUF):
                pltpu.make_async_copy(href.at[idxv.at[0]], rows.at[p].at[bb],
                                      gsem).wait()

        def start_scatters(p):
            for bb in range(NBUF):
                for j in range(BS // 16):
                    sdst[p, bb, pl.ds(j * 16, 16)] = (
                        idxv[6 * p + NBUF + bb, pl.ds(j * 16, 16)])
            for bb in range(NBUF):
                pltpu.async_copy(rows.at[p].at[bb],
                                 acc.at[sdst.at[p].at[bb]], ssem, add=True)

        def drain_scatters(p):
            for bb in range(NBUF):
                pltpu.make_async_copy(rows.at[p].at[bb],
                                      acc.at[sdst.at[p].at[bb]], ssem).wait()

        fetch_idx(0, 0)
        fetch_idx(1, 1)
        wait_idx(0)
        gathers(0)
        wait_idx(1)
        gathers(1)
        wait_gathers(0)
        start_scatters(0)
        fetch_idx(2, 0)

        def outer(o, _):
            for p in range(2):
                g = 2 * o + 1 + p
                q = (1 + p) % 2
                wait_idx(1 - q)
                drain_scatters(1 - q)
                gathers(1 - q)
                wait_gathers(q)
                start_scatters(q)
                fetch_idx(g + 2, q)
            return 0

        lax.fori_loop(0, (G - 4) // 2, outer, 0)
        wait_idx(0)
        drain_scatters(0)
        gathers(0)
        wait_gathers(1)
        start_scatters(1)
        fetch_idx(G - 1, 1)

        wait_idx(1)
        drain_scatters(1)
        gathers(1)
        wait_gathers(0)
        start_scatters(0)

        drain_scatters(0)
        wait_gathers(1)
        start_scatters(1)
        drain_scatters(1)

        plsc.subcore_barrier()
        pltpu.sync_copy(acc.at[pl.ds(row_lo, ROWS_PER_TILE)],
                        oref.at[pl.ds(row_lo, ROWS_PER_TILE)])
        plsc.subcore_barrier()

    for e, iref in enumerate((i0, i1, i2)):
        for cval in range(2):
            @pl.when(cid == cval)
            def _(cval=cval, e=e, iref=iref):
                one_pass(hrefs[cval], outs[e * 2 + cval], iref)


def _self_body(h, ws, s0, s1, s2):
    hblk = h[...]
    for e, sref in enumerate((s0, s1, s2)):
        sref[...] = jnp.dot(hblk, ws[e], preferred_element_type=jnp.float32)


def _tc_self(h, ws):
    nblk = N_PAD // TBLK
    return pl.pallas_call(
        _self_body,
        grid=(nblk,),
        in_specs=[pl.BlockSpec((TBLK, 128), lambda i: (i, 0)),
                  pl.BlockSpec((NE, 128, 128), lambda i: (0, 0, 0))],
        out_specs=[pl.BlockSpec((TBLK, 128), lambda i: (i, 0))
                   for _ in range(NE)],
        out_shape=[jax.ShapeDtypeStruct((N_PAD, 128), jnp.float32)
                   for _ in range(NE)],
    )(h, ws)


def _conv_body(s0, s1, s2, a00, a01, a10, a11, a20, a21,
               g0, g1, g2, wn, bs, out_ref, st_ref, sacc, *, act):
    i = pl.program_id(0)
    sb = (s0, s1, s2)
    ab = ((a00, a01), (a10, a11), (a20, a21))
    db = (g0, g1, g2)
    acc = jnp.zeros((TBLK, 128), jnp.float32)
    for e in range(NE):
        dg = db[e][...]
        deg = dg[0, :, 0:1] + dg[1, :, 0:1]
        inv = 1.0 / jnp.maximum(deg, 1.0)
        t = sb[e][...]
        for half in range(2):
            hn = ab[e][half][...].astype(jnp.float32) * inv
            t += jnp.dot(hn, wn[e, pl.ds(half * HW, HW), :],
                         preferred_element_type=jnp.float32)
        t += bs[pl.ds(e, 1), :]
        if act:
            t = jnp.maximum(t, 0.0)
        acc += t
    rows = i * TBLK + lax.broadcasted_iota(jnp.int32, (TBLK, 1), 0)
    acc = jnp.where(rows < N, acc, 0.0)
    out_ref[...] = acc
    st = jnp.concatenate(
        [jnp.sum(acc, axis=0, keepdims=True),
         jnp.sum(acc * acc, axis=0, keepdims=True)], axis=0)

    @pl.when(i == 0)
    def _():
        sacc[...] = st

    @pl.when(i > 0)
    def _():
        sacc[...] += st

    @pl.when(i == N_PAD // TBLK - 1)
    def _():
        st_ref[...] = sacc[...]


def _tc_conv(selfs, aggs, degs, wn, bias, act):
    nblk = N_PAD // TBLK
    sspec = pl.BlockSpec((TBLK, 128), lambda i: (i, 0))
    aspec = pl.BlockSpec((TBLK, HW), lambda i: (i, 0))
    dspec = pl.BlockSpec((2, TBLK, 32), lambda i: (0, i, 0))
    body = functools.partial(_conv_body, act=act)
    return pl.pallas_call(
        body,
        grid=(nblk,),
        in_specs=([sspec] * NE + [aspec] * (NE * 2) + [dspec] * NE
                  + [pl.BlockSpec((NE, 128, 128), lambda i: (0, 0, 0)),
                     pl.BlockSpec((NE, 128), lambda i: (0, 0))]),
        out_specs=[pl.BlockSpec((TBLK, 128), lambda i: (i, 0)),
                   pl.BlockSpec((2, 128), lambda i: (0, 0))],
        out_shape=[jax.ShapeDtypeStruct((N_PAD, 128), jnp.float32),
                   jax.ShapeDtypeStruct((2, 128), jnp.float32)],
        scratch_shapes=[pltpu.VMEM((2, 128), jnp.float32)],
    )(*selfs, *aggs, *degs, wn, bias)


def _bn_body(x_ref, st_ref, g_ref, b_ref, *out_refs, chunked):
    i = pl.program_id(0)
    st = st_ref[...]
    mean = st[0:1, :] / N
    var = st[1:2, :] / N - mean * mean
    scale = g_ref[...] / jnp.sqrt(var + 1e-5)
    shift = b_ref[...] - mean * scale
    y = x_ref[...] * scale + shift
    rows = i * TBLK + lax.broadcasted_iota(jnp.int32, (TBLK, 1), 0)
    y = jnp.where(rows < N, y, 0.0)
    out_refs[0][...] = y
    if chunked:
        yh = y.astype(jnp.bfloat16)
        out_refs[1][...] = yh[:, :HW]
        out_refs[2][...] = yh[:, HW:]


def _tc_bn(out, stats, g, b, chunked):
    nblk = N_PAD // TBLK
    if chunked:
        out_specs = [pl.BlockSpec((TBLK, 128), lambda i: (i, 0)),
                     pl.BlockSpec((TBLK, HW), lambda i: (i, 0)),
                     pl.BlockSpec((TBLK, HW), lambda i: (i, 0))]
        out_shape = [jax.ShapeDtypeStruct((N_PAD, 128), jnp.float32),
                     jax.ShapeDtypeStruct((N_PAD, HW), jnp.bfloat16),
                     jax.ShapeDtypeStruct((N_PAD, HW), jnp.bfloat16)]
    else:
        out_specs = [pl.BlockSpec((TBLK, 128), lambda i: (i, 0))]
        out_shape = [jax.ShapeDtypeStruct((N, 128), jnp.float32)]
    body = functools.partial(_bn_body, chunked=chunked)
    return pl.pallas_call(
        body,
        grid=(nblk,),
        in_specs=[pl.BlockSpec((TBLK, 128), lambda i: (i, 0)),
                  pl.BlockSpec((2, 128), lambda i: (0, 0)),
                  pl.BlockSpec((1, 128), lambda i: (0, 0)),
                  pl.BlockSpec((1, 128), lambda i: (0, 0))],
        out_specs=out_specs,
        out_shape=out_shape,
    )(out, stats, g, b)


def _prep_edge(ei):
    pad = E_PAD - E
    src = jnp.concatenate([ei[0], jnp.full((pad,), PAD_IDX, jnp.int32)])
    dst = jnp.concatenate([ei[1], jnp.full((pad,), PAD_IDX, jnp.int32)])
    src = src.reshape(16, G, NBUF, BS)
    dst = dst.reshape(16, G, NBUF, BS)
    return jnp.concatenate([src, dst], axis=2)


def kernel(x, edge_index_follows, edge_index_likes, edge_index_views,
           W_self, W_neigh, b, gamma, beta):
    eidx = [_prep_edge(e) for e in
            (edge_index_follows, edge_index_likes, edge_index_views)]
    zer32 = jnp.zeros((ROWS_PER_TILE, 32), jnp.float32)
    zer16 = jnp.zeros((ROWS_PER_TILE, HW), jnp.bfloat16)
    ones = jnp.ones((BS, 32), jnp.float32)

    degs = _sc_deg(eidx[0], eidx[1], eidx[2], ones, zer32)

    h = jnp.pad(x, ((0, N_PAD - N), (0, 0)))
    h16 = h.astype(jnp.bfloat16)
    halves = [h16[:, :HW], h16[:, HW:]]

    for l in range(L):
        aggs = _sc_agg(halves[0], halves[1],
                       eidx[0], eidx[1], eidx[2], zer16)
        selfs = _tc_self(h, W_self[l])
        out, stats = _tc_conv(selfs, aggs, degs, W_neigh[l],
                              b[l], act=(l < L - 1))
        res = _tc_bn(out, stats, gamma[l][None, :], beta[l][None, :],
                     chunked=(l < L - 1))
        if l < L - 1:
            h = res[0]
            halves = [res[1], res[2]]
        else:
            return res[0]


# speedup vs baseline: 2.5680x; 1.0393x over previous
import functools

import jax
import jax.numpy as jnp
from jax import lax
from jax.experimental import pallas as pl
from jax.experimental.pallas import tpu as pltpu
from jax.experimental.pallas import tpu_sc as plsc

N = 50000
D = 128
E = 200000
NE = 3
L = 3

HW = 64
N_PAD = 50176
ROWS_PER_TILE = N_PAD // 16
TBLK = 256

BS = 128
NBUF = 3
G = 34
NB = G * NBUF
E_PAD = 16 * NB * BS
PAD_IDX = N

_mesh = plsc.VectorSubcoreMesh(core_axis_name="c", subcore_axis_name="s")
_sc_params = pltpu.CompilerParams(use_tc_tiling_on_sc=False)


@functools.partial(
    pl.kernel,
    mesh=_mesh,
    compiler_params=_sc_params,
    out_type=[jax.ShapeDtypeStruct((2, N_PAD, 32), jnp.float32)
              for _ in range(NE)],
    scratch_types=[
        pltpu.VMEM((12, BS), jnp.int32),
        pltpu.VMEM((2, NBUF, BS), jnp.int32),
        pltpu.VMEM((BS, 32), jnp.float32),
        pltpu.SemaphoreType.DMA,
        pltpu.SemaphoreType.DMA,
        pltpu.VMEM_SHARED((N_PAD, 32), jnp.float32),
    ],
)
def _sc_deg(i0, i1, i2, ones_h, zer_h, o0, o1, o2,
            idxv, sdst, onesv, isem, ssem, acc):
    cid = lax.axis_index("c")
    sid = lax.axis_index("s")
    row_lo = sid * ROWS_PER_TILE
    pltpu.sync_copy(ones_h, onesv)
    GH = G // 2
    for e, (iref, oref) in enumerate(((i0, o0), (i1, o1), (i2, o2))):
        tidx = iref.at[sid]
        pltpu.sync_copy(zer_h, acc.at[pl.ds(row_lo, ROWS_PER_TILE)])
        plsc.subcore_barrier()
        base_g = cid * GH

        def fetch_idx(j, p, tidx=tidx):
            pltpu.async_copy(tidx.at[base_g + j], idxv.at[pl.ds(6 * p, 6)],
                             isem)

        def wait_idx(p, tidx=tidx):
            pltpu.make_async_copy(tidx.at[0], idxv.at[pl.ds(6 * p, 6)],
                                  isem).wait()

        def start_scatters(p):
            for bb in range(NBUF):
                for jj in range(BS // 16):
                    sdst[p, bb, pl.ds(jj * 16, 16)] = (
                        idxv[6 * p + NBUF + bb, pl.ds(jj * 16, 16)])
            for bb in range(NBUF):
                pltpu.async_copy(onesv, acc.at[sdst.at[p].at[bb]], ssem,
                                 add=True)

        def drain_scatters(p):
            for bb in range(NBUF):
                pltpu.make_async_copy(onesv, acc.at[sdst.at[p].at[bb]],
                                      ssem).wait()

        fetch_idx(0, 0)
        fetch_idx(1, 1)
        wait_idx(0)
        start_scatters(0)
        fetch_idx(2, 0)

        def outer(o, _):
            for p in range(2):
                j = 2 * o + 1 + p
                q = (1 + p) % 2
                wait_idx(q)
                start_scatters(q)
                drain_scatters(1 - q)
                fetch_idx(j + 2, q)
            return 0

        lax.fori_loop(0, (GH - 3) // 2, outer, 0)
        wait_idx(1)
        start_scatters(1)
        drain_scatters(0)
        wait_idx(0)
        start_scatters(0)
        drain_scatters(1)
        drain_scatters(0)
        plsc.subcore_barrier()
        for cval in range(2):
            @pl.when(cid == cval)
            def _(oref=oref, cval=cval):
                pltpu.sync_copy(acc.at[pl.ds(row_lo, ROWS_PER_TILE)],
                                oref.at[cval].at[pl.ds(row_lo, ROWS_PER_TILE)])
        plsc.subcore_barrier()


@functools.partial(
    pl.kernel,
    mesh=_mesh,
    compiler_params=_sc_params,
    out_type=[jax.ShapeDtypeStruct((N_PAD, HW), jnp.bfloat16)
              for _ in range(NE * 2)],
    scratch_types=[
        pltpu.VMEM((12, BS), jnp.int32),
        pltpu.VMEM((2, NBUF, BS), jnp.int32),
        pltpu.VMEM((2, NBUF, BS, HW), jnp.bfloat16),
        pltpu.SemaphoreType.DMA,
        pltpu.SemaphoreType.DMA,
        pltpu.SemaphoreType.DMA,
        pltpu.VMEM_SHARED((N_PAD, HW), jnp.bfloat16),
    ],
)
def _sc_agg(h0, h1, i0, i1, i2, zer_h, *rest):
    outs = rest[:NE * 2]
    idxv, sdst, rows, isem, gsem, ssem, acc = rest[NE * 2:]
    hrefs = (h0, h1)
    cid = lax.axis_index("c")
    sid = lax.axis_index("s")
    row_lo = sid * ROWS_PER_TILE

    def one_pass(href, oref, iref):
        tidx = iref.at[sid]
        pltpu.sync_copy(zer_h, acc.at[pl.ds(row_lo, ROWS_PER_TILE)])
        plsc.subcore_barrier()

        def fetch_idx(g, p):
            pltpu.async_copy(tidx.at[g], idxv.at[pl.ds(6 * p, 6)], isem)

        def wait_idx(p):
            pltpu.make_async_copy(tidx.at[0], idxv.at[pl.ds(6 * p, 6)],
                                  isem).wait()

        def gathers(p):
            for bb in range(NBUF):
                pltpu.async_copy(href.at[idxv.at[6 * p + bb]],
                                 rows.at[p].at[bb], gsem)

        def wait_gathers(p):
            for bb in range(NBUF):
                pltpu.make_async_copy(href.at[idxv.at[0]], rows.at[p].at[bb],
                                      gsem).wait()

        def start_scatters(p):
            for bb in range(NBUF):
                for j in range(BS // 16):
                    sdst[p, bb, pl.ds(j * 16, 16)] = (
                        idxv[6 * p + NBUF + bb, pl.ds(j * 16, 16)])
            for bb in range(NBUF):
                pltpu.async_copy(rows.at[p].at[bb],
                                 acc.at[sdst.at[p].at[bb]], ssem, add=True)

        def drain_scatters(p):
            for bb in range(NBUF):
                pltpu.make_async_copy(rows.at[p].at[bb],
                                      acc.at[sdst.at[p].at[bb]], ssem).wait()

        fetch_idx(0, 0)
        fetch_idx(1, 1)
        wait_idx(0)
        gathers(0)
        wait_idx(1)
        gathers(1)
        wait_gathers(0)
        start_scatters(0)
        fetch_idx(2, 0)

        def outer(o, _):
            for p in range(2):
                g = 2 * o + 1 + p
                q = (1 + p) % 2
                wait_idx(1 - q)
                drain_scatters(1 - q)
                gathers(1 - q)
                wait_gathers(q)
                start_scatters(q)
                fetch_idx(g + 2, q)
            return 0

        lax.fori_loop(0, (G - 4) // 2, outer, 0)
        wait_idx(0)
        drain_scatters(0)
        gathers(0)
        wait_gathers(1)
        start_scatters(1)
        fetch_idx(G - 1, 1)

        wait_idx(1)
        drain_scatters(1)
        gathers(1)
        wait_gathers(0)
        start_scatters(0)

        drain_scatters(0)
        wait_gathers(1)
        start_scatters(1)
        drain_scatters(1)

        plsc.subcore_barrier()
        pltpu.sync_copy(acc.at[pl.ds(row_lo, ROWS_PER_TILE)],
                        oref.at[pl.ds(row_lo, ROWS_PER_TILE)])
        plsc.subcore_barrier()

    for e, iref in enumerate((i0, i1, i2)):
        for cval in range(2):
            @pl.when(cid == cval)
            def _(cval=cval, e=e, iref=iref):
                one_pass(hrefs[cval], outs[e * 2 + cval], iref)


def _bn_coeffs(st, gm, bt):
    mean = st[0:1, :] / N
    var = st[1:2, :] / N - mean * mean
    sv = gm / jnp.sqrt(var + 1e-5)
    tv = bt - mean * sv
    return sv, tv


def _self_body(o, ws, st, gm, bt, s0, s1, s2):
    sv, tv = _bn_coeffs(st[...], gm[...], bt[...])
    hblk = o[...] * sv + tv
    for e, sref in enumerate((s0, s1, s2)):
        sref[...] = jnp.dot(hblk, ws[e], preferred_element_type=jnp.float32)


def _tc_self(o, ws, st, gm, bt):
    nblk = N_PAD // TBLK
    return pl.pallas_call(
        _self_body,
        grid=(nblk,),
        in_specs=[pl.BlockSpec((TBLK, 128), lambda i: (i, 0)),
                  pl.BlockSpec((NE, 128, 128), lambda i: (0, 0, 0)),
                  pl.BlockSpec((2, 128), lambda i: (0, 0)),
                  pl.BlockSpec((1, 128), lambda i: (0, 0)),
                  pl.BlockSpec((1, 128), lambda i: (0, 0))],
        out_specs=[pl.BlockSpec((TBLK, 128), lambda i: (i, 0))
                   for _ in range(NE)],
        out_shape=[jax.ShapeDtypeStruct((N_PAD, 128), jnp.float32)
                   for _ in range(NE)],
    )(o, ws, st, gm, bt)


def _conv_body(s0, s1, s2, a00, a01, a10, a11, a20, a21,
               g0, g1, g2, wn, bs, stp, gmp, btp,
               out_ref, st_ref, *rest, act, chunked):
    i = pl.program_id(0)
    sacc = rest[-1]
    sb = (s0, s1, s2)
    ab = ((a00, a01), (a10, a11), (a20, a21))
    db = (g0, g1, g2)
    sv, tv = _bn_coeffs(stp[...], gmp[...], btp[...])
    acc = jnp.zeros((TBLK, 128), jnp.float32)
    for e in range(NE):
        dg = db[e][...]
        deg = dg[0, :, 0:1] + dg[1, :, 0:1]
        inv = 1.0 / jnp.maximum(deg, 1.0)
        zdeg = deg * inv
        t = sb[e][...]
        for half in range(2):
            hn = (ab[e][half][...].astype(jnp.float32) * inv
                  * sv[:, half * HW:(half + 1) * HW])
            t += jnp.dot(hn, wn[e, pl.ds(half * HW, HW), :],
                         preferred_element_type=jnp.float32)
        t += zdeg * jnp.dot(tv, wn[e], preferred_element_type=jnp.float32)
        t += bs[pl.ds(e, 1), :]
        if act:
            t = jnp.maximum(t, 0.0)
        acc += t
    rows = i * TBLK + lax.broadcasted_iota(jnp.int32, (TBLK, 1), 0)
    acc = jnp.where(rows < N, acc, 0.0)
    out_ref[...] = acc
    if chunked:
        oh = acc.astype(jnp.bfloat16)
        rest[0][...] = oh[:, :HW]
        rest[1][...] = oh[:, HW:]
    st = jnp.concatenate(
        [jnp.sum(acc, axis=0, keepdims=True),
         jnp.sum(acc * acc, axis=0, keepdims=True)], axis=0)

    @pl.when(i == 0)
    def _():
        sacc[...] = st

    @pl.when(i > 0)
    def _():
        sacc[...] += st

    @pl.when(i == N_PAD // TBLK - 1)
    def _():
        st_ref[...] = sacc[...]


def _tc_conv(selfs, aggs, degs, wn, bias, stp, gmp, btp, act, chunked):
    nblk = N_PAD // TBLK
    sspec = pl.BlockSpec((TBLK, 128), lambda i: (i, 0))
    aspec = pl.BlockSpec((TBLK, HW), lambda i: (i, 0))
    dspec = pl.BlockSpec((2, TBLK, 32), lambda i: (0, i, 0))
    out_specs = [pl.BlockSpec((TBLK, 128), lambda i: (i, 0)),
                 pl.BlockSpec((2, 128), lambda i: (0, 0))]
    out_shape = [jax.ShapeDtypeStruct((N_PAD, 128), jnp.float32),
                 jax.ShapeDtypeStruct((2, 128), jnp.float32)]
    if chunked:
        out_specs += [pl.BlockSpec((TBLK, HW), lambda i: (i, 0))] * 2
        out_shape += [jax.ShapeDtypeStruct((N_PAD, HW), jnp.bfloat16)] * 2
    body = functools.partial(_conv_body, act=act, chunked=chunked)
    return pl.pallas_call(
        body,
        grid=(nblk,),
        in_specs=([sspec] * NE + [aspec] * (NE * 2) + [dspec] * NE
                  + [pl.BlockSpec((NE, 128, 128), lambda i: (0, 0, 0)),
                     pl.BlockSpec((NE, 128), lambda i: (0, 0)),
                     pl.BlockSpec((2, 128), lambda i: (0, 0)),
                     pl.BlockSpec((1, 128), lambda i: (0, 0)),
                     pl.BlockSpec((1, 128), lambda i: (0, 0))]),
        out_specs=out_specs,
        out_shape=out_shape,
        scratch_shapes=[pltpu.VMEM((2, 128), jnp.float32)],
    )(*selfs, *aggs, *degs, wn, bias, stp, gmp, btp)


def _bn_body(x_ref, st_ref, g_ref, b_ref, *out_refs, chunked):
    i = pl.program_id(0)
    st = st_ref[...]
    mean = st[0:1, :] / N
    var = st[1:2, :] / N - mean * mean
    scale = g_ref[...] / jnp.sqrt(var + 1e-5)
    shift = b_ref[...] - mean * scale
    y = x_ref[...] * scale + shift
    rows = i * TBLK + lax.broadcasted_iota(jnp.int32, (TBLK, 1), 0)
    y = jnp.where(rows < N, y, 0.0)
    out_refs[0][...] = y
    if chunked:
        yh = y.astype(jnp.bfloat16)
        out_refs[1][...] = yh[:, :HW]
        out_refs[2][...] = yh[:, HW:]


def _tc_bn(out, stats, g, b, chunked):
    nblk = N_PAD // TBLK
    if chunked:
        out_specs = [pl.BlockSpec((TBLK, 128), lambda i: (i, 0)),
                     pl.BlockSpec((TBLK, HW), lambda i: (i, 0)),
                     pl.BlockSpec((TBLK, HW), lambda i: (i, 0))]
        out_shape = [jax.ShapeDtypeStruct((N_PAD, 128), jnp.float32),
                     jax.ShapeDtypeStruct((N_PAD, HW), jnp.bfloat16),
                     jax.ShapeDtypeStruct((N_PAD, HW), jnp.bfloat16)]
    else:
        out_specs = [pl.BlockSpec((TBLK, 128), lambda i: (i, 0))]
        out_shape = [jax.ShapeDtypeStruct((N, 128), jnp.float32)]
    body = functools.partial(_bn_body, chunked=chunked)
    return pl.pallas_call(
        body,
        grid=(nblk,),
        in_specs=[pl.BlockSpec((TBLK, 128), lambda i: (i, 0)),
                  pl.BlockSpec((2, 128), lambda i: (0, 0)),
                  pl.BlockSpec((1, 128), lambda i: (0, 0)),
                  pl.BlockSpec((1, 128), lambda i: (0, 0))],
        out_specs=out_specs,
        out_shape=out_shape,
    )(out, stats, g, b)


def _prep_edge(ei):
    pad = E_PAD - E
    src = jnp.concatenate([ei[0], jnp.full((pad,), PAD_IDX, jnp.int32)])
    dst = jnp.concatenate([ei[1], jnp.full((pad,), PAD_IDX, jnp.int32)])
    src = src.reshape(16, G, NBUF, BS)
    dst = dst.reshape(16, G, NBUF, BS)
    return jnp.concatenate([src, dst], axis=2)


def kernel(x, edge_index_follows, edge_index_likes, edge_index_views,
           W_self, W_neigh, b, gamma, beta):
    eidx = [_prep_edge(e) for e in
            (edge_index_follows, edge_index_likes, edge_index_views)]
    zer32 = jnp.zeros((ROWS_PER_TILE, 32), jnp.float32)
    zer16 = jnp.zeros((ROWS_PER_TILE, HW), jnp.bfloat16)
    ones = jnp.ones((BS, 32), jnp.float32)

    degs = _sc_deg(eidx[0], eidx[1], eidx[2], ones, zer32)

    o = jnp.pad(x, ((0, N_PAD - N), (0, 0)))
    o16 = o.astype(jnp.bfloat16)
    halves = [o16[:, :HW], o16[:, HW:]]
    stp = jnp.stack([jnp.zeros((128,), jnp.float32),
                     jnp.full((128,), N * (1.0 - 1e-5), jnp.float32)])
    gmp = jnp.ones((1, 128), jnp.float32)
    btp = jnp.zeros((1, 128), jnp.float32)

    for l in range(L):
        aggs = _sc_agg(halves[0], halves[1],
                       eidx[0], eidx[1], eidx[2], zer16)
        selfs = _tc_self(o, W_self[l], stp, gmp, btp)
        res = _tc_conv(selfs, aggs, degs, W_neigh[l], b[l],
                       stp, gmp, btp, act=(l < L - 1), chunked=(l < L - 1))
        o, stats = res[0], res[1]
        if l < L - 1:
            halves = [res[2], res[3]]
        stp = stats
        gmp = gamma[l][None, :]
        btp = beta[l][None, :]
    return _tc_bn(o, stats, gamma[L - 1][None, :], beta[L - 1][None, :],
                  chunked=False)[0]
